# baseline (speedup 1.0000x reference)
import functools

import jax
import jax.numpy as jnp
from jax.experimental import pallas as pl
from jax.experimental.pallas import tpu as pltpu

_BLK = 1000
_HALO = 8


def _interleave_cols(wf, wb, sr, sz, sn):
    h = wf.shape[0] // 3
    return jnp.concatenate(
        [sr * wf[0:h], sr * wb[0:h],
         sz * wf[h:2 * h], sz * wb[h:2 * h],
         sn * wf[2 * h:3 * h], sn * wb[2 * h:3 * h]], axis=0)


def _interleave_vec(bf, bb):
    h = bf.shape[1] // 3
    return (jnp.concatenate([bf[:, 0:h], bb[:, 0:h]], axis=1),
            jnp.concatenate([bf[:, h:2 * h], bb[:, h:2 * h]], axis=1),
            jnp.concatenate([bf[:, 2 * h:3 * h], bb[:, 2 * h:3 * h]], axis=1))


def _prep_body(w1f_ref, b1fi_ref, b1fh_ref, w1b_ref, b1bi_ref, b1bh_ref,
               w2f_ref, b2fi_ref, b2fh_ref, w2b_ref, b2bi_ref, b2bh_ref,
               g1w_ref, g1b_ref, g2w_ref, g2b_ref, fw_ref, fb_ref,
               wt1_ref, ball1_ref, cn1_ref, wt2_ref, ball2_ref, cn2_ref,
               w12_ref, c1_ref, c2_ref, c12_ref):
    def gru_fold(wf, bfi, bfh, wb, bbi, bbh, in_scale, wt_ref, ball_ref,
                 cn_ref):
        wt_ref[...] = _interleave_cols(
            wf * in_scale, wb * in_scale, 0.5, -0.5, 1.0).astype(jnp.bfloat16)
        bir, biz, bin_ = _interleave_vec(bfi, bbi)
        bhr, bhz, bhn = _interleave_vec(bfh, bbh)
        ball_ref[...] = jnp.concatenate(
            [0.5 * (bir + bhr), -0.5 * (biz + bhz), bin_ + 0.5 * bhn], axis=1)
        cn_ref[...] = 0.5 * bhn

    gru_fold(w1f_ref[...], b1fi_ref[...], b1fh_ref[...],
             w1b_ref[...], b1bi_ref[...], b1bh_ref[...], 1.0,
             wt1_ref, ball1_ref, cn1_ref)
    gru_fold(w2f_ref[...], b2fi_ref[...], b2fh_ref[...],
             w2b_ref[...], b2bi_ref[...], b2bh_ref[...], 0.5,
             wt2_ref, ball2_ref, cn2_ref)

    g2f = jnp.dot(g2w_ref[...], fw_ref[...],
                  preferred_element_type=jnp.float32)
    w12_ref[...] = (0.5 / 9.0) * jnp.dot(g1w_ref[...], g2f,
                                         preferred_element_type=jnp.float32)
    c1 = jnp.dot(g1b_ref[...], g2f, preferred_element_type=jnp.float32)
    c2 = jnp.dot(g2b_ref[...], fw_ref[...],
                 preferred_element_type=jnp.float32) + fb_ref[...]
    c1_ref[...] = c1
    c2_ref[...] = c2
    c12_ref[...] = c1 + c2


def _shift_sum(y):
    yp = jnp.pad(y, ((1, 1), (0, 0)))
    return yp[:-2] + yp[1:-1] + yp[2:]


def _gru(xm, w_ref, ball_ref, cn_ref):
    h2 = 128
    g = jax.lax.dot_general(xm.astype(jnp.bfloat16), w_ref[...],
                            (((1,), (1,)), ((), ())),
                            preferred_element_type=jnp.float32)
    g = g + ball_ref[...]
    rt = jnp.tanh(g[:, 0:h2])
    zt = jnp.tanh(g[:, h2:2 * h2])
    n = jnp.tanh(g[:, 2 * h2:3 * h2] + cn_ref[...] * rt)
    return n * (1.0 + zt)


def _fused_body(x_ref, top_ref, bot_ref,
                w1_ref, ball1_ref, cn1_ref,
                w2_ref, ball2_ref, cn2_ref,
                w12_ref, c1_ref, c2_ref, c12_ref,
                out_ref, a_scr_ref, *, n_real, blk, halo, i_max):
    m = blk + 2 * halo

    xe = jnp.concatenate([top_ref[...], x_ref[...], bot_ref[...]], axis=0)
    seq1 = _gru(xe, w1_ref, ball1_ref, cn1_ref)
    seq2 = _gru(seq1, w2_ref, ball2_ref, cn2_ref)

    a = jnp.dot(seq2, w12_ref[...], preferred_element_type=jnp.float32)

    i = pl.program_id(0)
    interior = jnp.logical_and(i > 0, i <= i_max)

    @pl.when(interior)
    def _():
        a_scr_ref[...] = a
        a0 = a_scr_ref[pl.ds(halo, blk), :]
        am1 = a_scr_ref[pl.ds(halo - 1, blk), :]
        ap1 = a_scr_ref[pl.ds(halo + 1, blk), :]
        am2 = a_scr_ref[pl.ds(halo - 2, blk), :]
        ap2 = a_scr_ref[pl.ds(halo + 2, blk), :]
        s1 = a0 + (am1 + ap1)
        s2 = a0 + (am2 + ap2)
        out_ref[...] = (s1 + s1) + (s2 + c12_ref[...])

    @pl.when(jnp.logical_not(interior))
    def _():
        i0 = i * blk - halo
        idx = i0 + jax.lax.broadcasted_iota(jnp.int32, (m, 1), 0)
        valid = (idx >= 0) & (idx < n_real)
        end = (idx == 0) | (idx == n_real - 1)
        dinv = jnp.where(end, jax.lax.rsqrt(2.0), jax.lax.rsqrt(3.0))
        dv = jnp.where(valid, dinv, 0.0).astype(jnp.float32)
        dv3 = dv * 3.0
        t1 = dv3 * _shift_sum(a * dv3) + c1_ref[...]
        t2 = dv * _shift_sum(t1 * dv) + c2_ref[...]
        out_ref[...] = t2[halo:halo + blk]


@jax.jit
def kernel(x, g1f_wih, g1f_whh, g1f_bih, g1f_bhh, g1b_wih, g1b_whh, g1b_bih, g1b_bhh,
           g2f_wih, g2f_whh, g2f_bih, g2f_bhh, g2b_wih, g2b_whh, g2b_bih, g2b_bhh,
           gcn1_W, gcn1_b, gcn2_W, gcn2_b, fc_W, fc_b):
    n, d = x.shape
    odim = fc_W.shape[1]

    blk = _BLK
    halo = _HALO
    nb = -(-n // blk)
    npad = nb * blk
    if npad != n:
        x = jnp.pad(x, ((0, npad - n), (0, 0)))


    h3 = g1f_wih.shape[0]
    h2d = 2 * (h3 // 3) * 3
    prep_out = pl.pallas_call(
        _prep_body,
        out_shape=[
            jax.ShapeDtypeStruct((h2d, d), jnp.bfloat16),
            jax.ShapeDtypeStruct((1, h2d), jnp.float32),
            jax.ShapeDtypeStruct((1, 2 * h3 // 3), jnp.float32),
            jax.ShapeDtypeStruct((h2d, 2 * h3 // 3), jnp.bfloat16),
            jax.ShapeDtypeStruct((1, h2d), jnp.float32),
            jax.ShapeDtypeStruct((1, 2 * h3 // 3), jnp.float32),
            jax.ShapeDtypeStruct((gcn1_W.shape[0], odim), jnp.float32),
            jax.ShapeDtypeStruct((1, odim), jnp.float32),
            jax.ShapeDtypeStruct((1, odim), jnp.float32),
            jax.ShapeDtypeStruct((1, odim), jnp.float32),
        ],
    )(g1f_wih, g1f_bih[None, :], g1f_bhh[None, :],
      g1b_wih, g1b_bih[None, :], g1b_bhh[None, :],
      g2f_wih, g2f_bih[None, :], g2f_bhh[None, :],
      g2b_wih, g2b_bih[None, :], g2b_bhh[None, :],
      gcn1_W, gcn1_b[None, :], gcn2_W, gcn2_b[None, :],
      fc_W, fc_b[None, :])
    w1, ball1, cn1, w2, ball2, cn2, w12, c1, c2, c12 = prep_out

    i_max = (n - 1 - halo) // blk - 1

    def full(a):
        return pl.BlockSpec(a.shape, lambda i: (0,) * a.ndim)

    body = functools.partial(_fused_body, n_real=n, blk=blk, halo=halo,
                             i_max=i_max)
    out = pl.pallas_call(
        body,
        grid=(nb,),
        in_specs=[
            pl.BlockSpec((blk, d), lambda i: (i, 0)),
            pl.BlockSpec((halo, d),
                         lambda i: (jnp.maximum(i * (blk // halo) - 1, 0), 0)),
            pl.BlockSpec((halo, d),
                         lambda i: (jnp.minimum(i * (blk // halo) + blk // halo,
                                                npad // halo - 1), 0)),
            full(w1), full(ball1), full(cn1),
            full(w2), full(ball2), full(cn2),
            full(w12), full(c1), full(c2), full(c12),
        ],
        out_specs=pl.BlockSpec((blk, odim), lambda i: (i, 0)),
        out_shape=jax.ShapeDtypeStruct((npad, odim), jnp.float32),
        scratch_shapes=[pltpu.VMEM((blk + 2 * halo, odim), jnp.float32)],
    )(x, x, x,
      w1, ball1, cn1,
      w2, ball2, cn2,
      w12, c1, c2, c12)

    if npad != n:
        out = out[:n]
    return out


# speedup vs baseline: 1.3477x; 1.3477x over previous
import functools

import jax
import jax.numpy as jnp
from jax.experimental import pallas as pl
from jax.experimental.pallas import tpu as pltpu

_BLK = 2000
_HALO = 8


def _interleave_cols(wf, wb, sr, sz, sn):
    h = wf.shape[0] // 3
    return jnp.concatenate(
        [sr * wf[0:h], sr * wb[0:h],
         sz * wf[h:2 * h], sz * wb[h:2 * h],
         sn * wf[2 * h:3 * h], sn * wb[2 * h:3 * h]], axis=0)


def _interleave_vec(bf, bb):
    h = bf.shape[1] // 3
    return (jnp.concatenate([bf[:, 0:h], bb[:, 0:h]], axis=1),
            jnp.concatenate([bf[:, h:2 * h], bb[:, h:2 * h]], axis=1),
            jnp.concatenate([bf[:, 2 * h:3 * h], bb[:, 2 * h:3 * h]], axis=1))


def _prep_body(w1f_ref, b1fi_ref, b1fh_ref, w1b_ref, b1bi_ref, b1bh_ref,
               w2f_ref, b2fi_ref, b2fh_ref, w2b_ref, b2bi_ref, b2bh_ref,
               g1w_ref, g1b_ref, g2w_ref, g2b_ref, fw_ref, fb_ref,
               wt1_ref, ball1_ref, cn1_ref, wt2_ref, ball2_ref, cn2_ref,
               w12_ref, c1_ref, c2_ref, c12_ref):
    def gru_fold(wf, bfi, bfh, wb, bbi, bbh, in_scale, wt_ref, ball_ref,
                 cn_ref):
        wt_ref[...] = _interleave_cols(
            wf * in_scale, wb * in_scale, 0.5, -0.5, 1.0).astype(jnp.bfloat16)
        bir, biz, bin_ = _interleave_vec(bfi, bbi)
        bhr, bhz, bhn = _interleave_vec(bfh, bbh)
        ball_ref[...] = jnp.concatenate(
            [0.5 * (bir + bhr), -0.5 * (biz + bhz), bin_ + 0.5 * bhn], axis=1)
        cn_ref[...] = 0.5 * bhn

    gru_fold(w1f_ref[...], b1fi_ref[...], b1fh_ref[...],
             w1b_ref[...], b1bi_ref[...], b1bh_ref[...], 1.0,
             wt1_ref, ball1_ref, cn1_ref)
    gru_fold(w2f_ref[...], b2fi_ref[...], b2fh_ref[...],
             w2b_ref[...], b2bi_ref[...], b2bh_ref[...], 0.5,
             wt2_ref, ball2_ref, cn2_ref)

    g2f = jnp.dot(g2w_ref[...], fw_ref[...],
                  preferred_element_type=jnp.float32)
    w12_ref[...] = (0.5 / 9.0) * jnp.dot(g1w_ref[...], g2f,
                                         preferred_element_type=jnp.float32)
    c1 = jnp.dot(g1b_ref[...], g2f, preferred_element_type=jnp.float32)
    c2 = jnp.dot(g2b_ref[...], fw_ref[...],
                 preferred_element_type=jnp.float32) + fb_ref[...]
    c1_ref[...] = c1
    c2_ref[...] = c2
    c12_ref[...] = c1 + c2


def _shift_sum(y):
    yp = jnp.pad(y, ((1, 1), (0, 0)))
    return yp[:-2] + yp[1:-1] + yp[2:]


def _gru(xm, w_ref, ball_ref, cn_ref):
    h2 = 128
    g = jax.lax.dot_general(xm.astype(jnp.bfloat16), w_ref[...],
                            (((1,), (1,)), ((), ())),
                            preferred_element_type=jnp.float32)
    g = g + ball_ref[...]
    rt = jnp.tanh(g[:, 0:h2])
    zt = jnp.tanh(g[:, h2:2 * h2])
    n = jnp.tanh(g[:, 2 * h2:3 * h2] + cn_ref[...] * rt)
    return n * (1.0 + zt)


def _fused_body(x_ref, top_ref, bot_ref,
                w1_ref, ball1_ref, cn1_ref,
                w2_ref, ball2_ref, cn2_ref,
                w12_ref, c1_ref, c2_ref, c12_ref,
                out_ref, a_scr_ref, *, n_real, blk, halo, i_max):
    m = blk + 2 * halo

    xe = jnp.concatenate([top_ref[...], x_ref[...], bot_ref[...]], axis=0)
    seq1 = _gru(xe, w1_ref, ball1_ref, cn1_ref)
    seq2 = _gru(seq1, w2_ref, ball2_ref, cn2_ref)

    a = jnp.dot(seq2, w12_ref[...], preferred_element_type=jnp.float32)

    i = pl.program_id(0)
    interior = jnp.logical_and(i > 0, i <= i_max)

    @pl.when(interior)
    def _():
        a_scr_ref[...] = a
        a0 = a_scr_ref[pl.ds(halo, blk), :]
        am1 = a_scr_ref[pl.ds(halo - 1, blk), :]
        ap1 = a_scr_ref[pl.ds(halo + 1, blk), :]
        am2 = a_scr_ref[pl.ds(halo - 2, blk), :]
        ap2 = a_scr_ref[pl.ds(halo + 2, blk), :]
        s1 = a0 + (am1 + ap1)
        s2 = a0 + (am2 + ap2)
        out_ref[...] = (s1 + s1) + (s2 + c12_ref[...])

    @pl.when(jnp.logical_not(interior))
    def _():
        i0 = i * blk - halo
        idx = i0 + jax.lax.broadcasted_iota(jnp.int32, (m, 1), 0)
        valid = (idx >= 0) & (idx < n_real)
        end = (idx == 0) | (idx == n_real - 1)
        dinv = jnp.where(end, jax.lax.rsqrt(2.0), jax.lax.rsqrt(3.0))
        dv = jnp.where(valid, dinv, 0.0).astype(jnp.float32)
        dv3 = dv * 3.0
        t1 = dv3 * _shift_sum(a * dv3) + c1_ref[...]
        t2 = dv * _shift_sum(t1 * dv) + c2_ref[...]
        out_ref[...] = t2[halo:halo + blk]


@jax.jit
def kernel(x, g1f_wih, g1f_whh, g1f_bih, g1f_bhh, g1b_wih, g1b_whh, g1b_bih, g1b_bhh,
           g2f_wih, g2f_whh, g2f_bih, g2f_bhh, g2b_wih, g2b_whh, g2b_bih, g2b_bhh,
           gcn1_W, gcn1_b, gcn2_W, gcn2_b, fc_W, fc_b):
    n, d = x.shape
    odim = fc_W.shape[1]

    blk = _BLK
    halo = _HALO
    nb = -(-n // blk)
    npad = nb * blk
    if npad != n:
        x = jnp.pad(x, ((0, npad - n), (0, 0)))


    h3 = g1f_wih.shape[0]
    h2d = 2 * (h3 // 3) * 3
    prep_out = pl.pallas_call(
        _prep_body,
        out_shape=[
            jax.ShapeDtypeStruct((h2d, d), jnp.bfloat16),
            jax.ShapeDtypeStruct((1, h2d), jnp.float32),
            jax.ShapeDtypeStruct((1, 2 * h3 // 3), jnp.float32),
            jax.ShapeDtypeStruct((h2d, 2 * h3 // 3), jnp.bfloat16),
            jax.ShapeDtypeStruct((1, h2d), jnp.float32),
            jax.ShapeDtypeStruct((1, 2 * h3 // 3), jnp.float32),
            jax.ShapeDtypeStruct((gcn1_W.shape[0], odim), jnp.float32),
            jax.ShapeDtypeStruct((1, odim), jnp.float32),
            jax.ShapeDtypeStruct((1, odim), jnp.float32),
            jax.ShapeDtypeStruct((1, odim), jnp.float32),
        ],
    )(g1f_wih, g1f_bih[None, :], g1f_bhh[None, :],
      g1b_wih, g1b_bih[None, :], g1b_bhh[None, :],
      g2f_wih, g2f_bih[None, :], g2f_bhh[None, :],
      g2b_wih, g2b_bih[None, :], g2b_bhh[None, :],
      gcn1_W, gcn1_b[None, :], gcn2_W, gcn2_b[None, :],
      fc_W, fc_b[None, :])
    w1, ball1, cn1, w2, ball2, cn2, w12, c1, c2, c12 = prep_out

    i_max = (n - 1 - halo) // blk - 1

    def full(a):
        return pl.BlockSpec(a.shape, lambda i: (0,) * a.ndim)

    body = functools.partial(_fused_body, n_real=n, blk=blk, halo=halo,
                             i_max=i_max)
    out = pl.pallas_call(
        body,
        grid=(nb,),
        in_specs=[
            pl.BlockSpec((blk, d), lambda i: (i, 0)),
            pl.BlockSpec((halo, d),
                         lambda i: (jnp.maximum(i * (blk // halo) - 1, 0), 0)),
            pl.BlockSpec((halo, d),
                         lambda i: (jnp.minimum(i * (blk // halo) + blk // halo,
                                                npad // halo - 1), 0)),
            full(w1), full(ball1), full(cn1),
            full(w2), full(ball2), full(cn2),
            full(w12), full(c1), full(c2), full(c12),
        ],
        out_specs=pl.BlockSpec((blk, odim), lambda i: (i, 0)),
        out_shape=jax.ShapeDtypeStruct((npad, odim), jnp.float32),
        scratch_shapes=[pltpu.VMEM((blk + 2 * halo, odim), jnp.float32)],
    )(x, x, x,
      w1, ball1, cn1,
      w2, ball2, cn2,
      w12, c1, c2, c12)

    if npad != n:
        out = out[:n]
    return out
